# baseline (speedup 1.0000x reference)
import functools

import jax
import jax.numpy as jnp
from jax.experimental import pallas as pl

_TOP_K = 2
_SCALING = 16.0 / 8.0
_EPAD = 128


def _molora_body(x_ref, base_ref, w1a_ref, b1_ref, w2_ref, b2_ref,
                 bf_ref, out_ref):
    x = x_ref[...]
    y = jnp.dot(x, w1a_ref[...], preferred_element_type=jnp.float32)
    h = y[:, :256] + b1_ref[...]
    h = h * jax.nn.sigmoid(h)
    xa = y[:, 256:320]
    lg = jnp.dot(h, w2_ref[...], preferred_element_type=jnp.float32)
    lg = lg + b2_ref[...]

    lane = jax.lax.broadcasted_iota(jnp.int32, lg.shape, 1)
    m1 = jnp.max(lg, axis=-1, keepdims=True)
    i1 = jnp.min(jnp.where(lg == m1, lane, _EPAD), axis=-1, keepdims=True)
    lgm = jnp.where(lane == i1, -1e30, lg)
    m2 = jnp.max(lgm, axis=-1, keepdims=True)
    i2 = jnp.min(jnp.where(lgm == m2, lane, _EPAD), axis=-1, keepdims=True)
    e2 = jnp.exp(m2 - m1)
    rden = _SCALING / (1.0 + e2)
    a1 = rden
    a2 = e2 * rden

    elane = jax.lax.broadcasted_iota(jnp.int32, xa.shape, 1) // 8
    w64 = (jnp.where(elane == i1, a1, 0.0)
           + jnp.where(elane == i2, a2, 0.0))

    out_ref[...] = base_ref[...] + jnp.dot(
        xa * w64, bf_ref[...], preferred_element_type=jnp.float32)


@functools.partial(jax.jit, static_argnames=("interpret",))
def _molora(x, base_output, A, Bm, W1, b1, W2, b2, interpret=False):
    B, S, D = x.shape
    E, _, R = A.shape
    H = W1.shape[1]
    T = B * S
    TILE = 2048

    x2 = x.reshape(T, D)
    base2 = base_output.reshape(T, D)
    af = jnp.transpose(A, (1, 0, 2)).reshape(D, E * R)
    w1a = jnp.concatenate([W1, af], axis=1)
    bf = Bm.reshape(E * R, D)
    w2p = jnp.zeros((H, _EPAD), jnp.float32).at[:, :E].set(W2)
    b2p = jnp.full((1, _EPAD), -1e30, jnp.float32).at[0, :E].set(b2)
    b1r = b1.reshape(1, H)

    grid = (T // TILE,)
    out = pl.pallas_call(
        _molora_body,
        grid=grid,
        in_specs=[
            pl.BlockSpec((TILE, D), lambda i: (i, 0)),
            pl.BlockSpec((TILE, D), lambda i: (i, 0)),
            pl.BlockSpec((D, H + E * R), lambda i: (0, 0)),
            pl.BlockSpec((1, H), lambda i: (0, 0)),
            pl.BlockSpec((H, _EPAD), lambda i: (0, 0)),
            pl.BlockSpec((1, _EPAD), lambda i: (0, 0)),
            pl.BlockSpec((E * R, D), lambda i: (0, 0)),
        ],
        out_specs=pl.BlockSpec((TILE, D), lambda i: (i, 0)),
        out_shape=jax.ShapeDtypeStruct((T, D), jnp.float32),
        interpret=interpret,
    )(x2, base2, w1a, b1r, w2p, b2p, bf)
    return out.reshape(B, S, D)


def kernel(x, base_output, A, Bm, W1, b1, W2, b2):
    return _molora(x, base_output, A, Bm, W1, b1, W2, b2)
